# baseline (speedup 1.0000x reference)
import functools

import jax
import jax.numpy as jnp
from jax import lax
from jax.experimental import pallas as pl
from jax.experimental.pallas import tpu as pltpu
from jax.experimental.pallas import tpu_sc as plsc

N = 10000
D = 128
DA = 144
E = 320000
NC, NS = 2, 16
NW = NC * NS
NBUF = 3
CH = 80
EPW = E // NW
CPW = EPW // CH
GTRIPS = CPW // NBUF - 1
RPT = 640
NP = NS * RPT


def _sc_aggregate(xa, ei, zeros):
  mesh = plsc.VectorSubcoreMesh(
      core_axis_name="c", subcore_axis_name="s",
      num_cores=NC, num_subcores=NS)

  @functools.partial(
      pl.kernel,
      name="rgcn_sc_aggregate",
      out_type=[jax.ShapeDtypeStruct((NP, DA), jnp.float32),
                jax.ShapeDtypeStruct((NP, DA), jnp.float32)],
      mesh=mesh,
      compiler_params=pltpu.CompilerParams(use_tc_tiling_on_sc=False),
      scratch_types=[
          [pltpu.VMEM((CH,), jnp.int32)] * NBUF,
          [pltpu.VMEM((CH,), jnp.int32)] * NBUF,
          [pltpu.VMEM((CH, DA), jnp.float32)] * NBUF,
          pltpu.VMEM_SHARED((NP, DA), jnp.float32),
          [pltpu.SemaphoreType.DMA] * NBUF,
          [pltpu.SemaphoreType.DMA] * NBUF,
          [pltpu.SemaphoreType.DMA] * NBUF,
      ],
  )
  def body(xa_hbm, ei_hbm, zeros_hbm, out0_hbm, out1_hbm,
           srcbs, dstbs, rows, acc_sh, semg, semsi, semd):
    cid = lax.axis_index("c")
    sid = lax.axis_index("s")
    wid = cid * NS + sid
    base = sid * RPT
    ebase = wid * EPW

    def src_at(c):
      return ei_hbm.at[0, pl.ds(ebase + c * CH, CH)]

    def dst_at(c):
      return ei_hbm.at[1, pl.ds(ebase + c * CH, CH)]

    def wait_rows(buf_v, sem):
      pltpu.make_async_copy(zeros_hbm.at[pl.ds(0, CH)], buf_v, sem).wait()

    def wait_idx(buf_v, sem):
      pltpu.make_async_copy(ei_hbm.at[0, pl.ds(0, CH)], buf_v, sem).wait()

    for k in range(NBUF):
      pltpu.async_copy(src_at(k), srcbs[k], semsi[k])
      pltpu.async_copy(dst_at(k), dstbs[k], semd[k])
    pltpu.sync_copy(zeros_hbm.at[pl.ds(base, RPT)],
                    acc_sh.at[pl.ds(base, RPT)])
    plsc.subcore_barrier()
    for k in range(NBUF):
      wait_idx(srcbs[k], semsi[k])
      pltpu.async_copy(xa_hbm.at[srcbs[k]], rows[k], semg[k])

    def step(k, a, refill):
      wait_rows(rows[k], semg[k])
      if refill:
        pltpu.async_copy(src_at(a + NBUF), srcbs[k], semsi[k])
      wait_idx(dstbs[k], semd[k])
      pltpu.sync_copy(rows[k], acc_sh.at[dstbs[k]], add=True)
      if refill:
        pltpu.async_copy(dst_at(a + NBUF), dstbs[k], semd[k])
        wait_idx(srcbs[k], semsi[k])
        pltpu.async_copy(xa_hbm.at[srcbs[k]], rows[k], semg[k])

    def rotation(g, carry):
      a0 = NBUF * g
      for k in range(NBUF):
        step(k, a0 + k, True)
      return carry

    lax.fori_loop(0, GTRIPS, rotation, 0)
    left = CPW - NBUF * GTRIPS
    for j in range(left):
      k = j % NBUF
      step(k, NBUF * GTRIPS + j, j < left - NBUF)
    plsc.subcore_barrier()

    @pl.when(cid == 0)
    def _():
      pltpu.sync_copy(acc_sh.at[pl.ds(base, RPT)],
                      out0_hbm.at[pl.ds(base, RPT)])

    @pl.when(cid == 1)
    def _():
      pltpu.sync_copy(acc_sh.at[pl.ds(base, RPT)],
                      out1_hbm.at[pl.ds(base, RPT)])

  return body(xa, ei, zeros)


def _tc_combine(x, p0, p1, wrT, wlT, b):
  BLK = 1280

  def body(x_ref, p0_ref, p1_ref, wr_ref, wl_ref, b_ref, o_ref):
    msum = p0_ref[:, :D] + p1_ref[:, :D]
    cnt = p0_ref[:, D:D + 1] + p1_ref[:, D:D + 1]
    agg = msum * (1.0 / jnp.maximum(cnt, 1.0))
    o_ref[...] = (
        jnp.dot(x_ref[...], wr_ref[...], preferred_element_type=jnp.float32)
        + jnp.dot(agg, wl_ref[...], preferred_element_type=jnp.float32)
        + b_ref[...])

  return pl.pallas_call(
      body,
      grid=(pl.cdiv(N, BLK),),
      in_specs=[
          pl.BlockSpec((BLK, D), lambda i: (i, 0)),
          pl.BlockSpec((BLK, DA), lambda i: (i, 0)),
          pl.BlockSpec((BLK, DA), lambda i: (i, 0)),
          pl.BlockSpec((D, D), lambda i: (0, 0)),
          pl.BlockSpec((D, D), lambda i: (0, 0)),
          pl.BlockSpec((1, D), lambda i: (0, 0)),
      ],
      out_specs=pl.BlockSpec((BLK, D), lambda i: (i, 0)),
      out_shape=jax.ShapeDtypeStruct((N, D), jnp.float32),
  )(x, p0, p1, wrT, wlT, b)


def kernel(x, edge_index, W_rel, W_root, b_root):
  xa = jnp.concatenate(
      [x, jnp.ones((N, 1), jnp.float32), jnp.zeros((N, DA - D - 1), jnp.float32)],
      axis=1)
  zeros = jnp.zeros((NP, DA), jnp.float32)
  p0, p1 = _sc_aggregate(xa, edge_index, zeros)
  return _tc_combine(x, p0, p1, W_root.T, W_rel.T, b_root.reshape(1, D))


# speedup vs baseline: 1.2331x; 1.2331x over previous
import functools

import jax
import jax.numpy as jnp
from jax import lax
from jax.experimental import pallas as pl
from jax.experimental.pallas import tpu as pltpu
from jax.experimental.pallas import tpu_sc as plsc

N = 10000
D = 128
E = 320000
NC, NS = 2, 16
NW = NC * NS
NBUF = 3
CH = 80
EPW = E // NW
CPW = EPW // CH
GTRIPS = CPW // NBUF - 1
RPT = 640
NP = NS * RPT


def _sc_aggregate(x, ei, zeros, zeros_c):
  mesh = plsc.VectorSubcoreMesh(
      core_axis_name="c", subcore_axis_name="s",
      num_cores=NC, num_subcores=NS)

  @functools.partial(
      pl.kernel,
      name="rgcn_sc_aggregate",
      out_type=[jax.ShapeDtypeStruct((NP, D), jnp.float32),
                jax.ShapeDtypeStruct((NP, D), jnp.float32),
                jax.ShapeDtypeStruct((NP, 16), jnp.float32),
                jax.ShapeDtypeStruct((NP, 16), jnp.float32)],
      mesh=mesh,
      compiler_params=pltpu.CompilerParams(use_tc_tiling_on_sc=False),
      scratch_types=[
          [pltpu.VMEM((CH,), jnp.int32)] * NBUF,
          [pltpu.VMEM((CH,), jnp.int32)] * NBUF,
          [pltpu.VMEM((CH, D), jnp.float32)] * NBUF,
          pltpu.VMEM((CH, 16), jnp.float32),
          pltpu.VMEM_SHARED((NP, D), jnp.float32),
          pltpu.VMEM_SHARED((NP, 16), jnp.float32),
          [pltpu.SemaphoreType.DMA] * NBUF,
          [pltpu.SemaphoreType.DMA] * NBUF,
          [pltpu.SemaphoreType.DMA] * NBUF,
      ],
  )
  def body(x_hbm, ei_hbm, zeros_hbm, zeros_c_hbm,
           out0_hbm, out1_hbm, cout0_hbm, cout1_hbm,
           srcbs, dstbs, rows, ones_v, acc_sh, cnt_sh, semg, semsi, semd):
    cid = lax.axis_index("c")
    sid = lax.axis_index("s")
    wid = cid * NS + sid
    base = sid * RPT
    ebase = wid * EPW

    def src_at(c):
      return ei_hbm.at[0, pl.ds(ebase + c * CH, CH)]

    def dst_at(c):
      return ei_hbm.at[1, pl.ds(ebase + c * CH, CH)]

    def wait_rows(buf_v, sem):
      pltpu.make_async_copy(zeros_hbm.at[pl.ds(0, CH)], buf_v, sem).wait()

    def wait_idx(buf_v, sem):
      pltpu.make_async_copy(ei_hbm.at[0, pl.ds(0, CH)], buf_v, sem).wait()

    for k in range(NBUF):
      pltpu.async_copy(src_at(k), srcbs[k], semsi[k])
      pltpu.async_copy(dst_at(k), dstbs[k], semd[k])
    pltpu.sync_copy(zeros_hbm.at[pl.ds(base, RPT)],
                    acc_sh.at[pl.ds(base, RPT)])
    pltpu.sync_copy(zeros_c_hbm.at[pl.ds(base, RPT)],
                    cnt_sh.at[pl.ds(base, RPT)])

    def fill_ones(r, carry):
      ones_v[r, :] = jnp.ones((16,), jnp.float32)
      return carry

    lax.fori_loop(0, CH, fill_ones, 0)
    plsc.subcore_barrier()
    for k in range(NBUF):
      wait_idx(srcbs[k], semsi[k])
      pltpu.async_copy(x_hbm.at[srcbs[k]], rows[k], semg[k])

    def step(k, a, refill):
      wait_rows(rows[k], semg[k])
      if refill:
        pltpu.async_copy(src_at(a + NBUF), srcbs[k], semsi[k])
      wait_idx(dstbs[k], semd[k])
      pltpu.sync_copy(rows[k], acc_sh.at[dstbs[k]], add=True)
      pltpu.sync_copy(ones_v, cnt_sh.at[dstbs[k]], add=True)
      if refill:
        pltpu.async_copy(dst_at(a + NBUF), dstbs[k], semd[k])
        wait_idx(srcbs[k], semsi[k])
        pltpu.async_copy(x_hbm.at[srcbs[k]], rows[k], semg[k])

    def rotation(g, carry):
      a0 = NBUF * g
      for k in range(NBUF):
        step(k, a0 + k, True)
      return carry

    lax.fori_loop(0, GTRIPS, rotation, 0)
    left = CPW - NBUF * GTRIPS
    for j in range(left):
      k = j % NBUF
      step(k, NBUF * GTRIPS + j, j < left - NBUF)
    plsc.subcore_barrier()

    @pl.when(cid == 0)
    def _():
      pltpu.sync_copy(acc_sh.at[pl.ds(base, RPT)],
                      out0_hbm.at[pl.ds(base, RPT)])
      pltpu.sync_copy(cnt_sh.at[pl.ds(base, RPT)],
                      cout0_hbm.at[pl.ds(base, RPT)])

    @pl.when(cid == 1)
    def _():
      pltpu.sync_copy(acc_sh.at[pl.ds(base, RPT)],
                      out1_hbm.at[pl.ds(base, RPT)])
      pltpu.sync_copy(cnt_sh.at[pl.ds(base, RPT)],
                      cout1_hbm.at[pl.ds(base, RPT)])

  return body(x, ei, zeros, zeros_c)


def _tc_combine(x, p0, p1, c0, c1, wrT, wlT, b):
  BLK = 1280

  def body(x_ref, p0_ref, p1_ref, c0_ref, c1_ref, wr_ref, wl_ref, b_ref,
           o_ref):
    msum = p0_ref[...] + p1_ref[...]
    cnt = c0_ref[:, :1] + c1_ref[:, :1]
    agg = msum * (1.0 / jnp.maximum(cnt, 1.0))
    o_ref[...] = (
        jnp.dot(x_ref[...], wr_ref[...], preferred_element_type=jnp.float32)
        + jnp.dot(agg, wl_ref[...], preferred_element_type=jnp.float32)
        + b_ref[...])

  return pl.pallas_call(
      body,
      grid=(pl.cdiv(N, BLK),),
      in_specs=[
          pl.BlockSpec((BLK, D), lambda i: (i, 0)),
          pl.BlockSpec((BLK, D), lambda i: (i, 0)),
          pl.BlockSpec((BLK, D), lambda i: (i, 0)),
          pl.BlockSpec((BLK, 16), lambda i: (i, 0)),
          pl.BlockSpec((BLK, 16), lambda i: (i, 0)),
          pl.BlockSpec((D, D), lambda i: (0, 0)),
          pl.BlockSpec((D, D), lambda i: (0, 0)),
          pl.BlockSpec((1, D), lambda i: (0, 0)),
      ],
      out_specs=pl.BlockSpec((BLK, D), lambda i: (i, 0)),
      out_shape=jax.ShapeDtypeStruct((N, D), jnp.float32),
  )(x, p0, p1, c0, c1, wrT, wlT, b)


def kernel(x, edge_index, W_rel, W_root, b_root):
  zeros = jnp.zeros((NP, D), jnp.float32)
  zeros_c = jnp.zeros((NP, 16), jnp.float32)
  p0, p1, c0, c1 = _sc_aggregate(x, edge_index, zeros, zeros_c)
  return _tc_combine(x, p0, p1, c0, c1, W_root.T, W_rel.T,
                     b_root.reshape(1, D))


# speedup vs baseline: 1.2483x; 1.0124x over previous
import functools

import jax
import jax.numpy as jnp
from jax import lax
from jax.experimental import pallas as pl
from jax.experimental.pallas import tpu as pltpu
from jax.experimental.pallas import tpu_sc as plsc

N = 10000
D = 128
E = 320000
NC, NS = 2, 16
NW = NC * NS
NBUF = 3
CH = 80
EPW = E // NW
CPW = EPW // CH
GTRIPS = CPW // NBUF - 1
RPT = 640
NP = NS * RPT


def _sc_aggregate(x, ei, zeros, zeros_c):
  mesh = plsc.VectorSubcoreMesh(
      core_axis_name="c", subcore_axis_name="s",
      num_cores=NC, num_subcores=NS)

  @functools.partial(
      pl.kernel,
      name="rgcn_sc_aggregate",
      out_type=[jax.ShapeDtypeStruct((NP, D), jnp.float32),
                jax.ShapeDtypeStruct((NP, D), jnp.float32),
                jax.ShapeDtypeStruct((NP, 16), jnp.float32),
                jax.ShapeDtypeStruct((NP, 16), jnp.float32)],
      mesh=mesh,
      compiler_params=pltpu.CompilerParams(use_tc_tiling_on_sc=False),
      scratch_types=[
          [pltpu.VMEM((CH,), jnp.int32)] * NBUF,
          [pltpu.VMEM((CH,), jnp.int32)] * NBUF,
          [pltpu.VMEM((CH, D), jnp.float32)] * NBUF,
          pltpu.VMEM((CH, 16), jnp.float32),
          pltpu.VMEM_SHARED((NP, D), jnp.float32),
          pltpu.VMEM_SHARED((NP, 16), jnp.float32),
          [pltpu.SemaphoreType.DMA] * NBUF,
          [pltpu.SemaphoreType.DMA] * NBUF,
          [pltpu.SemaphoreType.DMA] * NBUF,
      ],
  )
  def body(x_hbm, ei_hbm, zeros_hbm, zeros_c_hbm,
           out0_hbm, out1_hbm, cout0_hbm, cout1_hbm,
           srcbs, dstbs, rows, ones_v, acc_sh, cnt_sh, semg, semsi, semd):
    cid = lax.axis_index("c")
    sid = lax.axis_index("s")
    wid = cid * NS + sid
    base = sid * RPT
    ebase = wid * EPW

    def src_at(c):
      return ei_hbm.at[0, pl.ds(ebase + c * CH, CH)]

    def dst_at(c):
      return ei_hbm.at[1, pl.ds(ebase + c * CH, CH)]

    def wait_rows(buf_v, sem):
      pltpu.make_async_copy(zeros_hbm.at[pl.ds(0, CH)], buf_v, sem).wait()

    def wait_idx(buf_v, sem):
      pltpu.make_async_copy(ei_hbm.at[0, pl.ds(0, CH)], buf_v, sem).wait()

    for k in range(NBUF):
      pltpu.async_copy(src_at(k), srcbs[k], semsi[k])
      pltpu.async_copy(dst_at(k), dstbs[k], semd[k])
    for k in range(NBUF):
      wait_idx(srcbs[k], semsi[k])
      pltpu.async_copy(x_hbm.at[srcbs[k]], rows[k], semg[k])
    pltpu.sync_copy(zeros_hbm.at[pl.ds(base, RPT)],
                    acc_sh.at[pl.ds(base, RPT)])
    pltpu.sync_copy(zeros_c_hbm.at[pl.ds(base, RPT)],
                    cnt_sh.at[pl.ds(base, RPT)])

    def fill_ones(r, carry):
      ones_v[r, :] = jnp.ones((16,), jnp.float32)
      return carry

    lax.fori_loop(0, CH, fill_ones, 0)
    plsc.subcore_barrier()

    def step(k, a, refill):
      wait_rows(rows[k], semg[k])
      if refill:
        pltpu.async_copy(src_at(a + NBUF), srcbs[k], semsi[k])
      wait_idx(dstbs[k], semd[k])
      pltpu.sync_copy(rows[k], acc_sh.at[dstbs[k]], add=True)
      pltpu.sync_copy(ones_v, cnt_sh.at[dstbs[k]], add=True)
      if refill:
        pltpu.async_copy(dst_at(a + NBUF), dstbs[k], semd[k])
        wait_idx(srcbs[k], semsi[k])
        pltpu.async_copy(x_hbm.at[srcbs[k]], rows[k], semg[k])

    def rotation(g, carry):
      a0 = NBUF * g
      for k in range(NBUF):
        step(k, a0 + k, True)
      return carry

    lax.fori_loop(0, GTRIPS, rotation, 0)
    left = CPW - NBUF * GTRIPS
    for j in range(left):
      k = j % NBUF
      step(k, NBUF * GTRIPS + j, j < left - NBUF)
    plsc.subcore_barrier()

    @pl.when(cid == 0)
    def _():
      pltpu.sync_copy(acc_sh.at[pl.ds(base, RPT)],
                      out0_hbm.at[pl.ds(base, RPT)])
      pltpu.sync_copy(cnt_sh.at[pl.ds(base, RPT)],
                      cout0_hbm.at[pl.ds(base, RPT)])

    @pl.when(cid == 1)
    def _():
      pltpu.sync_copy(acc_sh.at[pl.ds(base, RPT)],
                      out1_hbm.at[pl.ds(base, RPT)])
      pltpu.sync_copy(cnt_sh.at[pl.ds(base, RPT)],
                      cout1_hbm.at[pl.ds(base, RPT)])

  return body(x, ei, zeros, zeros_c)


def _tc_combine(x, p0, p1, c0, c1, wrT, wlT, b):
  BLK = 1280

  def body(x_ref, p0_ref, p1_ref, c0_ref, c1_ref, wr_ref, wl_ref, b_ref,
           o_ref):
    msum = p0_ref[...] + p1_ref[...]
    cnt = c0_ref[:, :1] + c1_ref[:, :1]
    agg = msum * (1.0 / jnp.maximum(cnt, 1.0))
    o_ref[...] = (
        jnp.dot(x_ref[...], wr_ref[...], preferred_element_type=jnp.float32)
        + jnp.dot(agg, wl_ref[...], preferred_element_type=jnp.float32)
        + b_ref[...])

  return pl.pallas_call(
      body,
      grid=(pl.cdiv(N, BLK),),
      in_specs=[
          pl.BlockSpec((BLK, D), lambda i: (i, 0)),
          pl.BlockSpec((BLK, D), lambda i: (i, 0)),
          pl.BlockSpec((BLK, D), lambda i: (i, 0)),
          pl.BlockSpec((BLK, 16), lambda i: (i, 0)),
          pl.BlockSpec((BLK, 16), lambda i: (i, 0)),
          pl.BlockSpec((D, D), lambda i: (0, 0)),
          pl.BlockSpec((D, D), lambda i: (0, 0)),
          pl.BlockSpec((1, D), lambda i: (0, 0)),
      ],
      out_specs=pl.BlockSpec((BLK, D), lambda i: (i, 0)),
      out_shape=jax.ShapeDtypeStruct((N, D), jnp.float32),
  )(x, p0, p1, c0, c1, wrT, wlT, b)


def kernel(x, edge_index, W_rel, W_root, b_root):
  zeros = jnp.zeros((NP, D), jnp.float32)
  zeros_c = jnp.zeros((NP, 16), jnp.float32)
  p0, p1, c0, c1 = _sc_aggregate(x, edge_index, zeros, zeros_c)
  return _tc_combine(x, p0, p1, c0, c1, W_root.T, W_rel.T,
                     b_root.reshape(1, D))


# speedup vs baseline: 1.2570x; 1.0070x over previous
import functools

import jax
import jax.numpy as jnp
from jax import lax
from jax.experimental import pallas as pl
from jax.experimental.pallas import tpu as pltpu
from jax.experimental.pallas import tpu_sc as plsc

N = 10000
D = 128
E = 320000
NC, NS = 2, 16
NW = NC * NS
NBUF = 3
CH = 80
EPW = E // NW
CPW = EPW // CH
GTRIPS = CPW // NBUF - 1
RPT = 640
NP = NS * RPT


def _sc_aggregate(x, ei, zeros, zeros_c):
  mesh = plsc.VectorSubcoreMesh(
      core_axis_name="c", subcore_axis_name="s",
      num_cores=NC, num_subcores=NS)

  @functools.partial(
      pl.kernel,
      name="rgcn_sc_aggregate",
      out_type=[jax.ShapeDtypeStruct((NP, D), jnp.float32),
                jax.ShapeDtypeStruct((NP, D), jnp.float32),
                jax.ShapeDtypeStruct((NP, 16), jnp.float32),
                jax.ShapeDtypeStruct((NP, 16), jnp.float32)],
      mesh=mesh,
      compiler_params=pltpu.CompilerParams(use_tc_tiling_on_sc=False),
      scratch_types=[
          [pltpu.VMEM((CH,), jnp.int32)] * NBUF,
          [pltpu.VMEM((CH,), jnp.int32)] * NBUF,
          [pltpu.VMEM((CH, D), jnp.float32)] * NBUF,
          pltpu.VMEM((CH, 16), jnp.float32),
          pltpu.VMEM_SHARED((NP, D), jnp.float32),
          pltpu.VMEM_SHARED((NP, 16), jnp.float32),
          [pltpu.SemaphoreType.DMA] * NBUF,
          [pltpu.SemaphoreType.DMA] * NBUF,
          [pltpu.SemaphoreType.DMA] * NBUF,
      ],
  )
  def body(x_hbm, ei_hbm, zeros_hbm, zeros_c_hbm,
           out0_hbm, out1_hbm, cout0_hbm, cout1_hbm,
           srcbs, dstbs, rows, ones_v, acc_sh, cnt_sh, semg, semsi, semd):
    cid = lax.axis_index("c")
    sid = lax.axis_index("s")
    wid = cid * NS + sid
    base = sid * RPT
    ebase = wid * EPW

    def src_at(c):
      return ei_hbm.at[0, pl.ds(ebase + c * CH, CH)]

    def dst_at(c):
      return ei_hbm.at[1, pl.ds(ebase + c * CH, CH)]

    def wait_rows(buf_v, sem):
      pltpu.make_async_copy(zeros_hbm.at[pl.ds(0, CH)], buf_v, sem).wait()

    def wait_idx(buf_v, sem):
      pltpu.make_async_copy(ei_hbm.at[0, pl.ds(0, CH)], buf_v, sem).wait()

    for k in range(NBUF):
      pltpu.async_copy(src_at(k), srcbs[k], semsi[k])
      pltpu.async_copy(dst_at(k), dstbs[k], semd[k])
    for k in range(NBUF):
      wait_idx(srcbs[k], semsi[k])
      pltpu.async_copy(x_hbm.at[srcbs[k]], rows[k], semg[k])
    pltpu.sync_copy(zeros_hbm.at[pl.ds(base, RPT)],
                    acc_sh.at[pl.ds(base, RPT)])
    pltpu.sync_copy(zeros_c_hbm.at[pl.ds(base, RPT)],
                    cnt_sh.at[pl.ds(base, RPT)])

    def fill_ones(r, carry):
      ones_v[r, :] = jnp.ones((16,), jnp.float32)
      return carry

    lax.fori_loop(0, CH, fill_ones, 0)
    plsc.subcore_barrier()

    def step(k, a, refill):
      wait_rows(rows[k], semg[k])
      if refill:
        pltpu.async_copy(src_at(a + NBUF), srcbs[k], semsi[k])
      wait_idx(dstbs[k], semd[k])
      pltpu.sync_copy(rows[k], acc_sh.at[dstbs[k]], add=True)
      pltpu.sync_copy(ones_v, cnt_sh.at[dstbs[k]], add=True)
      if refill:
        pltpu.async_copy(dst_at(a + NBUF), dstbs[k], semd[k])
        wait_idx(srcbs[k], semsi[k])
        pltpu.async_copy(x_hbm.at[srcbs[k]], rows[k], semg[k])

    def rotation(g, carry):
      a0 = NBUF * g
      for k in range(NBUF):
        step(k, a0 + k, True)
      return carry

    lax.fori_loop(0, GTRIPS, rotation, 0)
    left = CPW - NBUF * GTRIPS
    for j in range(left):
      k = j % NBUF
      step(k, NBUF * GTRIPS + j, j < left - NBUF)
    plsc.subcore_barrier()

    @pl.when(cid == 0)
    def _():
      pltpu.sync_copy(acc_sh.at[pl.ds(base, RPT)],
                      out0_hbm.at[pl.ds(base, RPT)])
      pltpu.sync_copy(cnt_sh.at[pl.ds(base, RPT)],
                      cout0_hbm.at[pl.ds(base, RPT)])

    @pl.when(cid == 1)
    def _():
      pltpu.sync_copy(acc_sh.at[pl.ds(base, RPT)],
                      out1_hbm.at[pl.ds(base, RPT)])
      pltpu.sync_copy(cnt_sh.at[pl.ds(base, RPT)],
                      cout1_hbm.at[pl.ds(base, RPT)])

  return body(x, ei, zeros, zeros_c)


def _tc_combine(x, p0, p1, c0, c1, wrT, wlT, b):
  BLK = 2000

  def body(x_ref, p0_ref, p1_ref, c0_ref, c1_ref, wr_ref, wl_ref, b_ref,
           o_ref):
    msum = p0_ref[...] + p1_ref[...]
    cnt = c0_ref[:, :1] + c1_ref[:, :1]
    agg = msum * (1.0 / jnp.maximum(cnt, 1.0))
    o_ref[...] = (
        jnp.dot(x_ref[...], wr_ref[...], preferred_element_type=jnp.float32)
        + jnp.dot(agg, wl_ref[...], preferred_element_type=jnp.float32)
        + b_ref[...])

  return pl.pallas_call(
      body,
      grid=(pl.cdiv(N, BLK),),
      in_specs=[
          pl.BlockSpec((BLK, D), lambda i: (i, 0)),
          pl.BlockSpec((BLK, D), lambda i: (i, 0)),
          pl.BlockSpec((BLK, D), lambda i: (i, 0)),
          pl.BlockSpec((BLK, 16), lambda i: (i, 0)),
          pl.BlockSpec((BLK, 16), lambda i: (i, 0)),
          pl.BlockSpec((D, D), lambda i: (0, 0)),
          pl.BlockSpec((D, D), lambda i: (0, 0)),
          pl.BlockSpec((1, D), lambda i: (0, 0)),
      ],
      out_specs=pl.BlockSpec((BLK, D), lambda i: (i, 0)),
      out_shape=jax.ShapeDtypeStruct((N, D), jnp.float32),
  )(x, p0, p1, c0, c1, wrT, wlT, b)


def kernel(x, edge_index, W_rel, W_root, b_root):
  zeros = jnp.zeros((NP, D), jnp.float32)
  zeros_c = jnp.zeros((NP, 16), jnp.float32)
  p0, p1, c0, c1 = _sc_aggregate(x, edge_index, zeros, zeros_c)
  return _tc_combine(x, p0, p1, c0, c1, W_root.T, W_rel.T,
                     b_root.reshape(1, D))


# speedup vs baseline: 1.2619x; 1.0038x over previous
import functools

import jax
import jax.numpy as jnp
from jax import lax
from jax.experimental import pallas as pl
from jax.experimental.pallas import tpu as pltpu
from jax.experimental.pallas import tpu_sc as plsc

N = 10000
D = 128
E = 320000
NC, NS = 2, 16
NW = NC * NS
NBUF = 3
CH = 80
EPW = E // NW
CPW = EPW // CH
GTRIPS = CPW // NBUF - 1
RPT = 640
NP = NS * RPT


def _sc_aggregate(x, ei, zeros, zeros_c):
  mesh = plsc.VectorSubcoreMesh(
      core_axis_name="c", subcore_axis_name="s",
      num_cores=NC, num_subcores=NS)

  @functools.partial(
      pl.kernel,
      name="rgcn_sc_aggregate",
      out_type=[jax.ShapeDtypeStruct((NP, D), jnp.float32),
                jax.ShapeDtypeStruct((NP, D), jnp.float32),
                jax.ShapeDtypeStruct((NP, 16), jnp.float32),
                jax.ShapeDtypeStruct((NP, 16), jnp.float32)],
      mesh=mesh,
      compiler_params=pltpu.CompilerParams(use_tc_tiling_on_sc=False),
      scratch_types=[
          [pltpu.VMEM((CH,), jnp.int32)] * NBUF,
          [pltpu.VMEM((CH,), jnp.int32)] * NBUF,
          [pltpu.VMEM((CH, D), jnp.float32)] * NBUF,
          pltpu.VMEM((CH, 16), jnp.float32),
          pltpu.VMEM_SHARED((NP, D), jnp.float32),
          pltpu.VMEM_SHARED((NP, 16), jnp.float32),
          [pltpu.SemaphoreType.DMA] * NBUF,
          [pltpu.SemaphoreType.DMA] * NBUF,
          [pltpu.SemaphoreType.DMA] * NBUF,
          pltpu.SemaphoreType.DMA,
          pltpu.SemaphoreType.DMA,
      ],
  )
  def body(x_hbm, ei_hbm, zeros_hbm, zeros_c_hbm,
           out0_hbm, out1_hbm, cout0_hbm, cout1_hbm,
           srcbs, dstbs, rows, ones_v, acc_sh, cnt_sh, semg, semsi, semd,
           semfs, semos):
    cid = lax.axis_index("c")
    sid = lax.axis_index("s")
    wid = cid * NS + sid
    base = sid * RPT
    ebase = wid * EPW

    def src_at(c):
      return ei_hbm.at[0, pl.ds(ebase + c * CH, CH)]

    def dst_at(c):
      return ei_hbm.at[1, pl.ds(ebase + c * CH, CH)]

    def wait_rows(buf_v, sem):
      pltpu.make_async_copy(zeros_hbm.at[pl.ds(0, CH)], buf_v, sem).wait()

    def wait_idx(buf_v, sem):
      pltpu.make_async_copy(ei_hbm.at[0, pl.ds(0, CH)], buf_v, sem).wait()

    for k in range(NBUF):
      pltpu.async_copy(src_at(k), srcbs[k], semsi[k])
      pltpu.async_copy(dst_at(k), dstbs[k], semd[k])
    for k in range(NBUF):
      wait_idx(srcbs[k], semsi[k])
      pltpu.async_copy(x_hbm.at[srcbs[k]], rows[k], semg[k])
    pltpu.sync_copy(zeros_hbm.at[pl.ds(base, RPT)],
                    acc_sh.at[pl.ds(base, RPT)])
    pltpu.sync_copy(zeros_c_hbm.at[pl.ds(base, RPT)],
                    cnt_sh.at[pl.ds(base, RPT)])

    def fill_ones(r, carry):
      ones_v[r, :] = jnp.ones((16,), jnp.float32)
      return carry

    lax.fori_loop(0, CH, fill_ones, 0)
    plsc.subcore_barrier()

    def step(k, a, refill):
      wait_rows(rows[k], semg[k])
      if refill:
        pltpu.async_copy(src_at(a + NBUF), srcbs[k], semsi[k])
      wait_idx(dstbs[k], semd[k])
      cp_f = pltpu.async_copy(rows[k], acc_sh.at[dstbs[k]], semfs, add=True)
      cp_o = pltpu.async_copy(ones_v, cnt_sh.at[dstbs[k]], semos, add=True)
      cp_f.wait()
      if refill:
        wait_idx(srcbs[k], semsi[k])
        pltpu.async_copy(x_hbm.at[srcbs[k]], rows[k], semg[k])
      cp_o.wait()
      if refill:
        pltpu.async_copy(dst_at(a + NBUF), dstbs[k], semd[k])

    def rotation(g, carry):
      a0 = NBUF * g
      for k in range(NBUF):
        step(k, a0 + k, True)
      return carry

    lax.fori_loop(0, GTRIPS, rotation, 0)
    left = CPW - NBUF * GTRIPS
    for j in range(left):
      k = j % NBUF
      step(k, NBUF * GTRIPS + j, j < left - NBUF)
    plsc.subcore_barrier()

    @pl.when(cid == 0)
    def _():
      pltpu.sync_copy(acc_sh.at[pl.ds(base, RPT)],
                      out0_hbm.at[pl.ds(base, RPT)])
      pltpu.sync_copy(cnt_sh.at[pl.ds(base, RPT)],
                      cout0_hbm.at[pl.ds(base, RPT)])

    @pl.when(cid == 1)
    def _():
      pltpu.sync_copy(acc_sh.at[pl.ds(base, RPT)],
                      out1_hbm.at[pl.ds(base, RPT)])
      pltpu.sync_copy(cnt_sh.at[pl.ds(base, RPT)],
                      cout1_hbm.at[pl.ds(base, RPT)])

  return body(x, ei, zeros, zeros_c)


def _tc_combine(x, p0, p1, c0, c1, wrT, wlT, b):
  BLK = 2000

  def body(x_ref, p0_ref, p1_ref, c0_ref, c1_ref, wr_ref, wl_ref, b_ref,
           o_ref):
    msum = p0_ref[...] + p1_ref[...]
    cnt = c0_ref[:, :1] + c1_ref[:, :1]
    agg = msum * (1.0 / jnp.maximum(cnt, 1.0))
    o_ref[...] = (
        jnp.dot(x_ref[...], wr_ref[...], preferred_element_type=jnp.float32)
        + jnp.dot(agg, wl_ref[...], preferred_element_type=jnp.float32)
        + b_ref[...])

  return pl.pallas_call(
      body,
      grid=(pl.cdiv(N, BLK),),
      in_specs=[
          pl.BlockSpec((BLK, D), lambda i: (i, 0)),
          pl.BlockSpec((BLK, D), lambda i: (i, 0)),
          pl.BlockSpec((BLK, D), lambda i: (i, 0)),
          pl.BlockSpec((BLK, 16), lambda i: (i, 0)),
          pl.BlockSpec((BLK, 16), lambda i: (i, 0)),
          pl.BlockSpec((D, D), lambda i: (0, 0)),
          pl.BlockSpec((D, D), lambda i: (0, 0)),
          pl.BlockSpec((1, D), lambda i: (0, 0)),
      ],
      out_specs=pl.BlockSpec((BLK, D), lambda i: (i, 0)),
      out_shape=jax.ShapeDtypeStruct((N, D), jnp.float32),
  )(x, p0, p1, c0, c1, wrT, wlT, b)


def kernel(x, edge_index, W_rel, W_root, b_root):
  zeros = jnp.zeros((NP, D), jnp.float32)
  zeros_c = jnp.zeros((NP, 16), jnp.float32)
  p0, p1, c0, c1 = _sc_aggregate(x, edge_index, zeros, zeros_c)
  return _tc_combine(x, p0, p1, c0, c1, W_root.T, W_rel.T,
                     b_root.reshape(1, D))
